# baseline (speedup 1.0000x reference)
import functools

import jax
import jax.numpy as jnp
from jax.experimental import pallas as pl
from jax.experimental.pallas import tpu as pltpu

NUM_EXPERTS = 8
TOP_K = 2
HIDDEN = 1024
FFN = 4096
FBLK = 1024


def _moe_kernel(x_ref, gate_ref, w1_ref, w3_ref, w2_ref, out_ref, wmat_ref):
    e = pl.program_id(0)
    f = pl.program_id(1)

    @pl.when((e == 0) & (f == 0))
    def _routing():
        x = x_ref[...]
        logits = jnp.dot(x, gate_ref[...].T, preferred_element_type=jnp.float32)
        p = jax.nn.softmax(logits, axis=-1)
        cols = jax.lax.broadcasted_iota(jnp.int32, p.shape, 1)
        i1 = jnp.argmax(p, axis=-1)
        oh1 = (cols == i1[:, None])
        m1 = jnp.max(p, axis=-1, keepdims=True)
        p2 = jnp.where(oh1, -jnp.inf, p)
        i2 = jnp.argmax(p2, axis=-1)
        oh2 = (cols == i2[:, None])
        m2 = jnp.max(p2, axis=-1, keepdims=True)
        s = m1 + m2
        wmat_ref[...] = oh1 * (m1 / s) + oh2 * (m2 / s)
        out_ref[...] = jnp.zeros_like(out_ref)

    xb = x_ref[...].astype(jnp.bfloat16)
    w1b = w1_ref[0].astype(jnp.bfloat16)
    w3b = w3_ref[0].astype(jnp.bfloat16)
    h1 = jnp.dot(xb, w1b.T, preferred_element_type=jnp.float32)
    h3 = jnp.dot(xb, w3b.T, preferred_element_type=jnp.float32)
    h = jax.nn.silu(h1) * h3
    eoh = (jax.lax.broadcasted_iota(jnp.int32, (NUM_EXPERTS, 1), 0) == e)
    wcol = jnp.dot(wmat_ref[...], eoh.astype(jnp.float32),
                   preferred_element_type=jnp.float32)
    h = (h * wcol).astype(jnp.bfloat16)
    w2b = w2_ref[0, :, pl.ds(f * FBLK, FBLK)].astype(jnp.bfloat16)
    out_ref[...] += jnp.dot(h, w2b.T, preferred_element_type=jnp.float32)


@functools.partial(jax.jit, static_argnames=())
def kernel(hidden_states, gate_w, w1, w2, w3):
    b, s, d = hidden_states.shape
    x = hidden_states.reshape(-1, d)
    t = x.shape[0]
    nf = FFN // FBLK

    out = pl.pallas_call(
        _moe_kernel,
        grid=(NUM_EXPERTS, nf),
        in_specs=[
            pl.BlockSpec((t, HIDDEN), lambda e, f: (0, 0)),
            pl.BlockSpec((NUM_EXPERTS, HIDDEN), lambda e, f: (0, 0)),
            pl.BlockSpec((1, FBLK, HIDDEN), lambda e, f: (e, f, 0)),
            pl.BlockSpec((1, FBLK, HIDDEN), lambda e, f: (e, f, 0)),
            pl.BlockSpec((1, HIDDEN, FFN), lambda e, f: (e, 0, 0)),
        ],
        out_specs=pl.BlockSpec((t, HIDDEN), lambda e, f: (0, 0)),
        out_shape=jax.ShapeDtypeStruct((t, HIDDEN), jnp.float32),
        scratch_shapes=[pltpu.VMEM((t, NUM_EXPERTS), jnp.float32)],
    )(x, gate_w, w1, w3, w2)
    return out.reshape(b, s, d)


# speedup vs baseline: 1.1549x; 1.1549x over previous
import functools

import jax
import jax.numpy as jnp
from jax.experimental import pallas as pl
from jax.experimental.pallas import tpu as pltpu

NUM_EXPERTS = 8
TOP_K = 2
HIDDEN = 1024
FFN = 4096
FBLK = 1024
HALF = FBLK // 2


def _moe_kernel(x_ref, gate_ref, w1a_ref, w1b_ref, w3a_ref, w3b_ref,
                w2a_ref, w2b_ref, out_ref, wmat_ref):
    e = pl.program_id(0)
    f = pl.program_id(1)

    @pl.when((e == 0) & (f == 0))
    def _routing():
        x = x_ref[...]
        logits = jnp.dot(x, gate_ref[...].T, preferred_element_type=jnp.float32)
        p = jax.nn.softmax(logits, axis=-1)
        cols = jax.lax.broadcasted_iota(jnp.int32, p.shape, 1)
        i1 = jnp.argmax(p, axis=-1)
        oh1 = (cols == i1[:, None])
        m1 = jnp.max(p, axis=-1, keepdims=True)
        p2 = jnp.where(oh1, -jnp.inf, p)
        i2 = jnp.argmax(p2, axis=-1)
        oh2 = (cols == i2[:, None])
        m2 = jnp.max(p2, axis=-1, keepdims=True)
        s = m1 + m2
        wmat_ref[...] = oh1 * (m1 / s) + oh2 * (m2 / s)
        out_ref[...] = jnp.zeros_like(out_ref)

    xb = x_ref[...].astype(jnp.bfloat16)
    eoh = (jax.lax.broadcasted_iota(jnp.int32, (NUM_EXPERTS, 1), 0) == e)
    wcol = jnp.dot(wmat_ref[...], eoh.astype(jnp.float32),
                   preferred_element_type=jnp.float32)

    def half(w1_ref, w3_ref, w2_ref):
        w1b = w1_ref[0].astype(jnp.bfloat16)
        w3b = w3_ref[0].astype(jnp.bfloat16)
        h1 = jnp.dot(xb, w1b.T, preferred_element_type=jnp.float32)
        h3 = jnp.dot(xb, w3b.T, preferred_element_type=jnp.float32)
        h = (jax.nn.silu(h1) * h3 * wcol).astype(jnp.bfloat16)
        w2b = w2_ref[0].astype(jnp.bfloat16)
        return jnp.dot(h, w2b.T, preferred_element_type=jnp.float32)

    out_ref[...] += (half(w1a_ref, w3a_ref, w2a_ref)
                     + half(w1b_ref, w3b_ref, w2b_ref))


@functools.partial(jax.jit, static_argnames=())
def kernel(hidden_states, gate_w, w1, w2, w3):
    b, s, d = hidden_states.shape
    x = hidden_states.reshape(-1, d)
    t = x.shape[0]
    nf = FFN // FBLK

    ffn_a = pl.BlockSpec((1, HALF, HIDDEN), lambda e, f: (e, 2 * f, 0))
    ffn_b = pl.BlockSpec((1, HALF, HIDDEN), lambda e, f: (e, 2 * f + 1, 0))
    col_a = pl.BlockSpec((1, HIDDEN, HALF), lambda e, f: (e, 0, 2 * f))
    col_b = pl.BlockSpec((1, HIDDEN, HALF), lambda e, f: (e, 0, 2 * f + 1))

    out = pl.pallas_call(
        _moe_kernel,
        grid=(NUM_EXPERTS, nf),
        in_specs=[
            pl.BlockSpec((t, HIDDEN), lambda e, f: (0, 0)),
            pl.BlockSpec((NUM_EXPERTS, HIDDEN), lambda e, f: (0, 0)),
            ffn_a, ffn_b, ffn_a, ffn_b, col_a, col_b,
        ],
        out_specs=pl.BlockSpec((t, HIDDEN), lambda e, f: (0, 0)),
        out_shape=jax.ShapeDtypeStruct((t, HIDDEN), jnp.float32),
        scratch_shapes=[pltpu.VMEM((t, NUM_EXPERTS), jnp.float32)],
    )(x, gate_w, w1, w1, w3, w3, w2, w2)
    return out.reshape(b, s, d)


# speedup vs baseline: 1.1622x; 1.0063x over previous
import functools

import jax
import jax.numpy as jnp
from jax.experimental import pallas as pl
from jax.experimental.pallas import tpu as pltpu

NUM_EXPERTS = 8
TOP_K = 2
HIDDEN = 1024
FFN = 4096
FBLK = 1024
QUAR = FBLK // 4


def _moe_kernel(x_ref, gate_ref, w1a_ref, w1b_ref, w1c_ref, w1d_ref,
                w3a_ref, w3b_ref, w3c_ref, w3d_ref,
                w2a_ref, w2b_ref, w2c_ref, w2d_ref, out_ref, wmat_ref):
    e = pl.program_id(0)
    f = pl.program_id(1)

    @pl.when((e == 0) & (f == 0))
    def _routing():
        x = x_ref[...]
        logits = jnp.dot(x, gate_ref[...].T, preferred_element_type=jnp.float32)
        p = jax.nn.softmax(logits, axis=-1)
        cols = jax.lax.broadcasted_iota(jnp.int32, p.shape, 1)
        i1 = jnp.argmax(p, axis=-1)
        oh1 = (cols == i1[:, None])
        m1 = jnp.max(p, axis=-1, keepdims=True)
        p2 = jnp.where(oh1, -jnp.inf, p)
        i2 = jnp.argmax(p2, axis=-1)
        oh2 = (cols == i2[:, None])
        m2 = jnp.max(p2, axis=-1, keepdims=True)
        s = m1 + m2
        wmat_ref[...] = oh1 * (m1 / s) + oh2 * (m2 / s)
        out_ref[...] = jnp.zeros_like(out_ref)

    xb = x_ref[...].astype(jnp.bfloat16)
    eoh = (jax.lax.broadcasted_iota(jnp.int32, (NUM_EXPERTS, 1), 0) == e)
    wcol = jnp.dot(wmat_ref[...], eoh.astype(jnp.float32),
                   preferred_element_type=jnp.float32)

    def half(w1_ref, w3_ref, w2_ref):
        w1b = w1_ref[0].astype(jnp.bfloat16)
        w3b = w3_ref[0].astype(jnp.bfloat16)
        h1 = jnp.dot(xb, w1b.T, preferred_element_type=jnp.float32)
        h3 = jnp.dot(xb, w3b.T, preferred_element_type=jnp.float32)
        h = (jax.nn.silu(h1) * h3 * wcol).astype(jnp.bfloat16)
        w2b = w2_ref[0].astype(jnp.bfloat16)
        return jnp.dot(h, w2b.T, preferred_element_type=jnp.float32)

    out_ref[...] += (half(w1a_ref, w3a_ref, w2a_ref)
                     + half(w1b_ref, w3b_ref, w2b_ref)
                     + half(w1c_ref, w3c_ref, w2c_ref)
                     + half(w1d_ref, w3d_ref, w2d_ref))


@functools.partial(jax.jit, static_argnames=())
def kernel(hidden_states, gate_w, w1, w2, w3):
    b, s, d = hidden_states.shape
    x = hidden_states.reshape(-1, d)
    t = x.shape[0]
    nf = FFN // FBLK

    ffn = [pl.BlockSpec((1, QUAR, HIDDEN), (lambda k: (lambda e, f: (e, 4 * f + k, 0)))(k))
           for k in range(4)]
    col = [pl.BlockSpec((1, HIDDEN, QUAR), (lambda k: (lambda e, f: (e, 0, 4 * f + k)))(k))
           for k in range(4)]

    out = pl.pallas_call(
        _moe_kernel,
        grid=(NUM_EXPERTS, nf),
        in_specs=[
            pl.BlockSpec((t, HIDDEN), lambda e, f: (0, 0)),
            pl.BlockSpec((NUM_EXPERTS, HIDDEN), lambda e, f: (0, 0)),
            *ffn, *ffn, *col,
        ],
        out_specs=pl.BlockSpec((t, HIDDEN), lambda e, f: (0, 0)),
        out_shape=jax.ShapeDtypeStruct((t, HIDDEN), jnp.float32),
        scratch_shapes=[pltpu.VMEM((t, NUM_EXPERTS), jnp.float32)],
    )(x, gate_w, w1, w1, w1, w1, w3, w3, w3, w3, w2, w2, w2, w2)
    return out.reshape(b, s, d)
